# Initial kernel scaffold; baseline (speedup 1.0000x reference)
import jax
import jax.numpy as jnp
from jax.experimental import pallas as pl


def kernel(Z, thruster_x, hydro_x, buoy_x, ea_t, ea_h, ea_b, params):
    raise NotImplementedError("write your pallas kernel here")


# speedup vs baseline: 17.9815x; 17.9815x over previous
import functools

import jax
import jax.numpy as jnp
from jax.experimental import pallas as pl

_B = 65536
_N_THR = 8
_HID = 32
_N_OUT = 9
_BLK = 512


def _asp(beta, x):
    bx = beta * x
    sp = jnp.maximum(bx, 0.0) + jnp.log1p(jnp.exp(-jnp.abs(bx)))
    return sp / beta


def _ln(x, g, b):
    mu = jnp.mean(x, axis=-1, keepdims=True)
    xc = x - mu
    var = jnp.mean(xc * xc, axis=-1, keepdims=True)
    return xc * jax.lax.rsqrt(var + 1e-5) * g + b


def _dot(a, b):
    return jax.lax.dot_general(a, b, (((1,), (0,)), ((), ())),
                               preferred_element_type=jnp.float32)


def _mlp2_tail(p, h):
    h = _asp(p['beta1'], h)
    h = _ln(h, p['g1'], p['be1'])
    return _dot(h, p['W2']) + p['b2']


def _prep_mlp(p, splits, three=False):
    out = {}
    ws = []
    off = 0
    for s in splits:
        ws.append(p['W1'][off:off + s])
        off += s
    out['W1s'] = tuple(ws)
    for k in ('b1', 'g1', 'be1', 'b2'):
        out[k] = p[k].reshape(1, -1)
    out['beta1'] = p['beta1'].reshape(1, 1)
    out['W2'] = p['W2']
    if three:
        for k in ('g2', 'be2', 'b3'):
            out[k] = p[k].reshape(1, -1)
        out['beta2'] = p['beta2'].reshape(1, 1)
        out['W3'] = p['W3']
    return out


def _pignn_body(z_ref, thr_ref, hyd_ref, buo_ref, eat_ref, eah_ref, eab_ref,
                prep_refs, out_ref):
    P = jax.tree.map(lambda r: r[...], prep_refs)
    n = _BLK
    e = n * _N_THR

    z = z_ref[...]
    pe = P['enc']
    hull = _mlp2_tail(pe, _dot(z, pe['W1s'][0]) + pe['b1'])

    thr = thr_ref[...]
    eat = eat_ref[...]
    hyd = hyd_ref[...]
    eah = eah_ref[...]
    buo = buo_ref[...]
    eab = eab_ref[...]

    for li in range(2):
        L = P['layers'][li]
        pt = L['thr']
        hull_part = _dot(hull, pt['W1s'][2]) + pt['b1']
        h = _dot(thr, pt['W1s'][0]) + _dot(eat, pt['W1s'][1])
        h = h + jnp.broadcast_to(hull_part[:, None, :], (n, _N_THR, _HID)
                                 ).reshape(e, _HID)
        h = _asp(pt['beta1'], h)
        h = _ln(h, pt['g1'], pt['be1'])
        msg = _dot(h, pt['W2']) + pt['b2']
        agg = msg.reshape(n, _N_THR, _HID).sum(axis=1)
        ph = L['hyd']
        hh = (_dot(hyd, ph['W1s'][0]) + _dot(eah, ph['W1s'][1])
              + _dot(hull, ph['W1s'][2]) + ph['b1'])
        msg_h = _mlp2_tail(ph, hh)
        pb = L['buo']
        hb = (_dot(buo, pb['W1s'][0]) + _dot(eab, pb['W1s'][1])
              + _dot(hull, pb['W1s'][2]) + pb['b1'])
        msg_b = _mlp2_tail(pb, hb)
        pu = L['upd']
        hu = (_dot(hull, pu['W1s'][0]) + _dot(agg, pu['W1s'][1])
              + _dot(msg_h, pu['W1s'][2]) + _dot(msg_b, pu['W1s'][3])
              + pu['b1'])
        hull = _mlp2_tail(pu, hu)

    pr = P['readout']
    h = _dot(hull, pr['W1s'][0]) + pr['b1']
    h = _asp(pr['beta1'], h)
    h = _ln(h, pr['g1'], pr['be1'])
    h = _dot(h, pr['W2']) + pr['b2']
    h = _asp(pr['beta2'], h)
    h = _ln(h, pr['g2'], pr['be2'])
    delta = _dot(h, pr['W3']) + pr['b3']

    state = z[:, 0:_N_OUT]
    xh = delta + state
    cos = xh[:, 3:4]
    sin = xh[:, 4:5]
    d0 = delta[:, 0:1]
    d1 = delta[:, 1:2]
    xw = cos * d0 - sin * d1 + z[:, 0:1]
    yw = sin * d0 + cos * d1 + z[:, 1:2]
    col = jax.lax.broadcasted_iota(jnp.int32, (n, _N_OUT), 1)
    out_ref[...] = jnp.where(col == 0, xw, jnp.where(col == 1, yw, xh))


@jax.jit
def kernel(Z, thruster_x, hydro_x, buoy_x, ea_t, ea_h, ea_b, params):
    prep = {
        'enc': _prep_mlp(params['enc'], (14,)),
        'layers': [
            {
                'thr': _prep_mlp(lp['thr'], (8, 7, _HID)),
                'hyd': _prep_mlp(lp['hyd'], (4, 8, _HID)),
                'buo': _prep_mlp(lp['buo'], (6, 4, _HID)),
                'upd': _prep_mlp(lp['upd'], (_HID,) * 4),
            }
            for lp in params['layers']
        ],
        'readout': _prep_mlp(params['readout'], (_HID,), three=True),
    }

    n = _BLK
    grid = _B // n

    def row_spec(rows, cols):
        return pl.BlockSpec((rows, cols), lambda i: (i, 0))

    prep_specs = jax.tree.map(
        lambda a: pl.BlockSpec(a.shape, lambda i: (0,) * a.ndim), prep)

    return pl.pallas_call(
        _pignn_body,
        grid=(grid,),
        in_specs=[
            row_spec(n, 14),
            row_spec(n * _N_THR, 8),
            row_spec(n, 4),
            row_spec(n, 6),
            row_spec(n * _N_THR, 7),
            row_spec(n, 8),
            row_spec(n, 4),
            prep_specs,
        ],
        out_specs=row_spec(n, _N_OUT),
        out_shape=jax.ShapeDtypeStruct((_B, _N_OUT), jnp.float32),
    )(Z, thruster_x, hydro_x, buoy_x, ea_t, ea_h, ea_b, prep)


# speedup vs baseline: 23.9921x; 1.3343x over previous
import jax
import jax.numpy as jnp
from jax.experimental import pallas as pl

_B = 65536
_N_THR = 8
_HID = 32
_N_OUT = 9
_BLK = 1024


def _asp(beta, rbeta, x):
    bx = beta * x
    sp = jnp.maximum(bx, 0.0) + jnp.log1p(jnp.exp(-jnp.abs(bx)))
    return sp * rbeta


def _dot(a, b):
    return jax.lax.dot_general(a, b, (((1,), (0,)), ((), ())),
                               preferred_element_type=jnp.float32)


def _avg_stat(x, avg_bf):
    hi = x.astype(jnp.bfloat16)
    lo = (x - hi.astype(jnp.float32)).astype(jnp.bfloat16)
    return _dot(hi, avg_bf) + _dot(lo, avg_bf)


def _ln_grouped(a, avg, g_t, be_t):
    mu = _avg_stat(a, avg)
    asq = _avg_stat(a * a, avg)
    var = asq - mu * mu
    return (a - mu) * jax.lax.rsqrt(var + 1e-5) * g_t + be_t


def _bd(w, copies):
    r, c = w.shape
    out = jnp.zeros((r * copies, c * copies), jnp.float32)
    for i in range(copies):
        out = out.at[i * r:(i + 1) * r, i * c:(i + 1) * c].set(w)
    return out


def _split_rows(w, sizes):
    out = []
    off = 0
    for s in sizes:
        out.append(w[off:off + s])
        off += s
    return tuple(out)


def _prep(params):
    p = {}
    enc = params['enc']
    p['enc'] = {
        'W1': enc['W1'], 'b1': enc['b1'].reshape(1, -1),
        'beta1': enc['beta1'].reshape(1, 1),
        'rbeta1': (1.0 / enc['beta1']).reshape(1, 1),
        'g1': enc['g1'].reshape(1, -1), 'be1': enc['be1'].reshape(1, -1),
        'W2': enc['W2'], 'b2': enc['b2'].reshape(1, -1),
    }
    layers = []
    for lp in params['layers']:
        t = lp['thr']
        wt, we, wh = _split_rows(t['W1'], (8, 7, _HID))
        thr = {
            'W1t': _bd(wt, _N_THR),
            'W1e': _bd(we, _N_THR),
            'W1h': jnp.tile(wh, (1, _N_THR)),
            'b1': jnp.tile(t['b1'].reshape(1, -1), (1, _N_THR)),
            'beta1': t['beta1'].reshape(1, 1),
            'rbeta1': (1.0 / t['beta1']).reshape(1, 1),
            'g1': jnp.tile(t['g1'].reshape(1, -1), (1, _N_THR)),
            'be1': jnp.tile(t['be1'].reshape(1, -1), (1, _N_THR)),
            'W2s': jnp.tile(t['W2'], (_N_THR, 1)),
            'b2x8': t['b2'].reshape(1, -1) * float(_N_THR),
        }
        h, b = lp['hyd'], lp['buo']
        hx, hea, hh = _split_rows(h['W1'], (4, 8, _HID))
        bx, bea, bh = _split_rows(b['W1'], (6, 4, _HID))
        z32 = jnp.zeros((_HID, _HID), jnp.float32)
        hb = {
            'W1hx': jnp.concatenate([hx, jnp.zeros((4, _HID), jnp.float32)], 1),
            'W1hea': jnp.concatenate([hea, jnp.zeros((8, _HID), jnp.float32)], 1),
            'W1bx': jnp.concatenate([jnp.zeros((6, _HID), jnp.float32), bx], 1),
            'W1bea': jnp.concatenate([jnp.zeros((4, _HID), jnp.float32), bea], 1),
            'W1h': jnp.concatenate([hh, bh], 1),
            'b1': jnp.concatenate([h['b1'], b['b1']]).reshape(1, -1),
            'beta': jnp.concatenate([
                jnp.full((_HID,), 1.0, jnp.float32) * h['beta1'],
                jnp.full((_HID,), 1.0, jnp.float32) * b['beta1']]).reshape(1, -1),
            'rbeta': jnp.concatenate([
                jnp.full((_HID,), 1.0, jnp.float32) / h['beta1'],
                jnp.full((_HID,), 1.0, jnp.float32) / b['beta1']]).reshape(1, -1),
            'g1': jnp.concatenate([h['g1'], b['g1']]).reshape(1, -1),
            'be1': jnp.concatenate([h['be1'], b['be1']]).reshape(1, -1),
            'W2': jnp.concatenate([
                jnp.concatenate([h['W2'], z32], 1),
                jnp.concatenate([z32, b['W2']], 1)], 0),
            'b2': jnp.concatenate([h['b2'], b['b2']]).reshape(1, -1),
        }
        u = lp['upd']
        uh, ua, um1, um2 = _split_rows(u['W1'], (_HID,) * 4)
        upd = {
            'W1h': uh, 'W1a': ua,
            'W1m': jnp.concatenate([um1, um2], 0),
            'b1': u['b1'].reshape(1, -1),
            'beta1': u['beta1'].reshape(1, 1),
            'rbeta1': (1.0 / u['beta1']).reshape(1, 1),
            'g1': u['g1'].reshape(1, -1), 'be1': u['be1'].reshape(1, -1),
            'W2': u['W2'], 'b2': u['b2'].reshape(1, -1),
        }
        layers.append({'thr': thr, 'hb': hb, 'upd': upd})
    r = params['readout']
    p['layers'] = layers
    j32 = jnp.full((_HID, _HID), 1.0 / _HID, jnp.bfloat16)
    p['avg32'] = j32
    p['avg64'] = _bd(j32, 2)
    p['avg256'] = _bd(j32, _N_THR)
    p['readout'] = {
        'W1': r['W1'], 'b1': r['b1'].reshape(1, -1),
        'beta1': r['beta1'].reshape(1, 1), 'rbeta1': (1.0 / r['beta1']).reshape(1, 1),
        'g1': r['g1'].reshape(1, -1), 'be1': r['be1'].reshape(1, -1),
        'W2': r['W2'], 'b2': r['b2'].reshape(1, -1),
        'beta2': r['beta2'].reshape(1, 1), 'rbeta2': (1.0 / r['beta2']).reshape(1, 1),
        'g2': r['g2'].reshape(1, -1), 'be2': r['be2'].reshape(1, -1),
        'W3': r['W3'], 'b3': r['b3'].reshape(1, -1),
    }
    return p


def _pignn_body(z_ref, thr_ref, hyd_ref, buo_ref, eat_ref, eah_ref, eab_ref,
                prep_refs, out_ref):
    P = jax.tree.map(lambda r: r[...], prep_refs)
    n = _BLK

    z = z_ref[...]
    pe = P['enc']
    a = _asp(pe['beta1'], pe['rbeta1'], _dot(z, pe['W1']) + pe['b1'])
    hull = _dot(_ln_grouped(a, P['avg32'], pe['g1'], pe['be1']), pe['W2']) + pe['b2']

    thr = thr_ref[...]
    eat = eat_ref[...]
    hyd = hyd_ref[...]
    eah = eah_ref[...]
    buo = buo_ref[...]
    eab = eab_ref[...]

    for li in range(2):
        L = P['layers'][li]
        pt = L['thr']
        pre = (_dot(thr, pt['W1t']) + _dot(eat, pt['W1e'])
               + _dot(hull, pt['W1h']) + pt['b1'])
        a = _asp(pt['beta1'], pt['rbeta1'], pre)
        y = _ln_grouped(a, P['avg256'], pt['g1'], pt['be1'])
        agg = _dot(y, pt['W2s']) + pt['b2x8']
        ph = L['hb']
        pre = (_dot(hyd, ph['W1hx']) + _dot(eah, ph['W1hea'])
               + _dot(buo, ph['W1bx']) + _dot(eab, ph['W1bea'])
               + _dot(hull, ph['W1h']) + ph['b1'])
        bx = ph['beta'] * pre
        a = (jnp.maximum(bx, 0.0) + jnp.log1p(jnp.exp(-jnp.abs(bx)))) * ph['rbeta']
        y = _ln_grouped(a, P['avg64'], ph['g1'], ph['be1'])
        msg_hb = _dot(y, ph['W2']) + ph['b2']
        pu = L['upd']
        pre = (_dot(hull, pu['W1h']) + _dot(agg, pu['W1a'])
               + _dot(msg_hb, pu['W1m']) + pu['b1'])
        a = _asp(pu['beta1'], pu['rbeta1'], pre)
        hull = _dot(_ln_grouped(a, P['avg32'], pu['g1'], pu['be1']), pu['W2']) + pu['b2']

    pr = P['readout']
    a = _asp(pr['beta1'], pr['rbeta1'], _dot(hull, pr['W1']) + pr['b1'])
    h = _dot(_ln_grouped(a, P['avg32'], pr['g1'], pr['be1']), pr['W2']) + pr['b2']
    a = _asp(pr['beta2'], pr['rbeta2'], h)
    delta = _dot(_ln_grouped(a, P['avg32'], pr['g2'], pr['be2']), pr['W3']) + pr['b3']

    state = z[:, 0:_N_OUT]
    xh = delta + state
    cos = xh[:, 3:4]
    sin = xh[:, 4:5]
    d0 = delta[:, 0:1]
    d1 = delta[:, 1:2]
    xw = cos * d0 - sin * d1 + z[:, 0:1]
    yw = sin * d0 + cos * d1 + z[:, 1:2]
    col = jax.lax.broadcasted_iota(jnp.int32, (n, _N_OUT), 1)
    out_ref[...] = jnp.where(col == 0, xw, jnp.where(col == 1, yw, xh))


@jax.jit
def kernel(Z, thruster_x, hydro_x, buoy_x, ea_t, ea_h, ea_b, params):
    prep = _prep(params)
    thr_f = thruster_x.reshape(_B, _N_THR * 8)
    eat_f = ea_t.reshape(_B, _N_THR * 7)

    n = _BLK
    grid = _B // n

    def row_spec(rows, cols):
        return pl.BlockSpec((rows, cols), lambda i: (i, 0))

    prep_specs = jax.tree.map(
        lambda a: pl.BlockSpec(a.shape, lambda i: (0,) * a.ndim), prep)

    return pl.pallas_call(
        _pignn_body,
        grid=(grid,),
        in_specs=[
            row_spec(n, 14),
            row_spec(n, _N_THR * 8),
            row_spec(n, 4),
            row_spec(n, 6),
            row_spec(n, _N_THR * 7),
            row_spec(n, 8),
            row_spec(n, 4),
            prep_specs,
        ],
        out_specs=row_spec(n, _N_OUT),
        out_shape=jax.ShapeDtypeStruct((_B, _N_OUT), jnp.float32),
    )(Z, thr_f, hydro_x, buoy_x, eat_f, ea_h, ea_b, prep)
